# baseline (speedup 1.0000x reference)
import functools

import jax
import jax.numpy as jnp
from jax import lax
from jax.experimental import pallas as pl
from jax.experimental.pallas import tpu as pltpu
from jax.experimental.pallas import tpu_sc as plsc

ROWS = 16384
COLS = 64
NUM_CORES = 2
NUM_SUBCORES = 16
NW = NUM_CORES * NUM_SUBCORES
RPW = ROWS // NW
GROUPS = RPW // 16
CCHUNK = COLS // 2


def _sc_body(spt_hbm, en_hbm, tab_hbm, out_hbm, spo_hbm,
             spt_v, tab16_v, t2_v, tab_v, out_v, semt, seme, sem0, sem1, semo):
    wid = lax.axis_index("s") * NUM_CORES + lax.axis_index("c")
    base = wid * RPW

    cpt = pltpu.async_copy(tab_hbm, tab16_v.at[pl.ds(0, 8)], semt)
    cp0 = pltpu.async_copy(
        spt_hbm.at[pl.ds(0, CCHUNK), pl.ds(base, RPW)],
        spt_v.at[pl.ds(0, CCHUNK)], sem0)
    cp1 = pltpu.async_copy(
        spt_hbm.at[pl.ds(CCHUNK, CCHUNK), pl.ds(base, RPW)],
        spt_v.at[pl.ds(CCHUNK, CCHUNK)], sem1)
    cpe = pltpu.async_copy(en_hbm.at[pl.ds(base, RPW)], out_v, seme)

    lane = lax.iota(jnp.int32, 16)

    cpt.wait()
    for j in range(4):
        ent = j * 16 + lane
        t2j = (plsc.load_gather(tab16_v, [(ent >> 3) & 7])
               + plsc.load_gather(tab16_v, [ent & 7]))
        t2_v[pl.ds(j * 16, 16)] = t2j
    for e in range(64):
        tab_v[pl.ds(e * 16, 16)] = plsc.load_gather(
            t2_v, [jnp.full((16,), e, jnp.int32)])

    def group(g, _):
        row0 = g * 16
        acc = out_v[pl.ds(row0, 16)]
        for c in range(0, COLS, 2):
            sp_a = spt_v[c, pl.ds(row0, 16)]
            sp_b = spt_v[c + 1, pl.ds(row0, 16)]
            idx = ((((sp_a << 3) | sp_b) << 4) + lane) & 1023
            acc = acc + plsc.load_gather(tab_v, [idx])
        out_v[pl.ds(row0, 16)] = acc
        return 0

    cpe.wait()
    cp0.wait()
    cp1.wait()
    cpo = pltpu.async_copy(spt_v, spo_hbm.at[:, pl.ds(base, RPW)], semo)
    lax.fori_loop(0, GROUPS, group, 0)

    pltpu.sync_copy(out_v, out_hbm.at[pl.ds(base, RPW)])
    cpo.wait()


@functools.partial(
    pl.kernel,
    out_type=(
        jax.ShapeDtypeStruct((ROWS,), jnp.float32),
        jax.ShapeDtypeStruct((COLS, ROWS), jnp.int32),
    ),
    mesh=plsc.VectorSubcoreMesh(core_axis_name="c", subcore_axis_name="s"),
    compiler_params=pltpu.CompilerParams(needs_layout_passes=False),
    scratch_types=[
        pltpu.VMEM((COLS, RPW), jnp.int32),
        pltpu.VMEM((16,), jnp.float32),
        pltpu.VMEM((64,), jnp.float32),
        pltpu.VMEM((1024,), jnp.float32),
        pltpu.VMEM((RPW,), jnp.float32),
        pltpu.SemaphoreType.DMA,
        pltpu.SemaphoreType.DMA,
        pltpu.SemaphoreType.DMA,
        pltpu.SemaphoreType.DMA,
        pltpu.SemaphoreType.DMA,
    ],
)
def _shift(spt_hbm, en_hbm, tab_hbm, out_hbm, spo_hbm,
           spt_v, tab16_v, t2_v, tab_v, out_v, semt, seme, sem0, sem1, semo):
    _sc_body(spt_hbm, en_hbm, tab_hbm, out_hbm, spo_hbm,
             spt_v, tab16_v, t2_v, tab_v, out_v, semt, seme, sem0, sem1, semo)


def kernel(species, energies, self_energies):
    spt = species.astype(jnp.int32).T
    shifted, spo = _shift(
        spt, energies.astype(jnp.float32), self_energies.astype(jnp.float32))
    return (spo.T.astype(species.dtype), shifted)


# speedup vs baseline: 1.0096x; 1.0096x over previous
import functools

import jax
import jax.numpy as jnp
from jax import lax
from jax.experimental import pallas as pl
from jax.experimental.pallas import tpu as pltpu
from jax.experimental.pallas import tpu_sc as plsc

ROWS = 16384
COLS = 64
NUM_CORES = 2
NUM_SUBCORES = 16
NW = NUM_CORES * NUM_SUBCORES
RPW = ROWS // NW
GROUPS = RPW // 16
CCHUNK = COLS // 2


def _sc_body(spt_hbm, en_hbm, tab_hbm, out_hbm, spo_hbm,
             spt_v, tab16_v, tab_v, out_v, semt, seme, sem0, sem1, semo):
    wid = lax.axis_index("s") * NUM_CORES + lax.axis_index("c")
    base = wid * RPW

    cpt = pltpu.async_copy(tab_hbm, tab16_v.at[pl.ds(0, 8)], semt)
    cp0 = pltpu.async_copy(
        spt_hbm.at[pl.ds(0, CCHUNK), pl.ds(base, RPW)],
        spt_v.at[pl.ds(0, CCHUNK)], sem0)
    cp1 = pltpu.async_copy(
        spt_hbm.at[pl.ds(CCHUNK, CCHUNK), pl.ds(base, RPW)],
        spt_v.at[pl.ds(CCHUNK, CCHUNK)], sem1)
    cpe = pltpu.async_copy(en_hbm.at[pl.ds(base, RPW)], out_v, seme)

    lane = lax.iota(jnp.int32, 16)

    cpt.wait()
    t16 = tab16_v[...]
    for e in range(64):
        val = t16[e >> 3] + t16[e & 7]
        tab_v[pl.ds(e * 16, 16)] = jnp.full((16,), val, jnp.float32)

    def group(g, _):
        row0 = g * 16
        acc = out_v[pl.ds(row0, 16)]
        for c in range(0, COLS, 2):
            sp_a = spt_v[c, pl.ds(row0, 16)]
            sp_b = spt_v[c + 1, pl.ds(row0, 16)]
            idx = ((((sp_a << 3) | sp_b) << 4) + lane) & 1023
            acc = acc + plsc.load_gather(tab_v, [idx])
        out_v[pl.ds(row0, 16)] = acc
        return 0

    cpe.wait()
    cp0.wait()
    cp1.wait()
    cpo = pltpu.async_copy(spt_v, spo_hbm.at[:, pl.ds(base, RPW)], semo)
    lax.fori_loop(0, GROUPS, group, 0)

    pltpu.sync_copy(out_v, out_hbm.at[pl.ds(base, RPW)])
    cpo.wait()


@functools.partial(
    pl.kernel,
    out_type=(
        jax.ShapeDtypeStruct((ROWS,), jnp.float32),
        jax.ShapeDtypeStruct((COLS, ROWS), jnp.int32),
    ),
    mesh=plsc.VectorSubcoreMesh(core_axis_name="c", subcore_axis_name="s"),
    compiler_params=pltpu.CompilerParams(needs_layout_passes=False),
    scratch_types=[
        pltpu.VMEM((COLS, RPW), jnp.int32),
        pltpu.VMEM((16,), jnp.float32),
        pltpu.VMEM((1024,), jnp.float32),
        pltpu.VMEM((RPW,), jnp.float32),
        pltpu.SemaphoreType.DMA,
        pltpu.SemaphoreType.DMA,
        pltpu.SemaphoreType.DMA,
        pltpu.SemaphoreType.DMA,
        pltpu.SemaphoreType.DMA,
    ],
)
def _shift(spt_hbm, en_hbm, tab_hbm, out_hbm, spo_hbm,
           spt_v, tab16_v, tab_v, out_v, semt, seme, sem0, sem1, semo):
    _sc_body(spt_hbm, en_hbm, tab_hbm, out_hbm, spo_hbm,
             spt_v, tab16_v, tab_v, out_v, semt, seme, sem0, sem1, semo)


def kernel(species, energies, self_energies):
    spt = species.astype(jnp.int32).T
    shifted, spo = _shift(
        spt, energies.astype(jnp.float32), self_energies.astype(jnp.float32))
    return (spo.T.astype(species.dtype), shifted)
